# baseline (speedup 1.0000x reference)
import jax
import jax.numpy as jnp
from jax import lax
from jax.experimental import pallas as pl
from jax.experimental.pallas import tpu as pltpu
from jax.experimental.pallas import tpu_sc as plsc

N = 10000
E = 320000
G = 64
D = 128

NC = 2
NS = 16
NW = NC * NS
CH = 80
EPW = 10240
EP = NW * EPW
NPAD = 10240
RPT = NPAD // NS
L = 16

_mesh = plsc.VectorSubcoreMesh(
    core_axis_name="c", subcore_axis_name="s", num_cores=NC, num_subcores=NS
)


DCH = 128


def _sc_deg_body(dst_hbm, degp_hbm, idx_v, buf_v, deg_sh):
    cid = lax.axis_index("c")
    sid = lax.axis_index("s")
    wid = sid * NC + cid
    base = wid * EPW

    def fill(val):
        def body(i, _):
            for j in range(D // L):
                buf_v[i, pl.ds(j * L, L)] = jnp.full((L,), val, jnp.float32)
            return 0
        return body

    lax.fori_loop(0, DCH, fill(0.0), 0)
    for k in range(RPT // DCH):
        pltpu.sync_copy(buf_v, deg_sh.at[pl.ds(sid * RPT + k * DCH, DCH)])
    plsc.subcore_barrier()
    lax.fori_loop(0, DCH, fill(1.0), 0)

    def chunk_body(ci, _):
        pltpu.sync_copy(dst_hbm.at[pl.ds(base + ci * DCH, DCH)], idx_v)
        pltpu.sync_copy(buf_v, deg_sh.at[idx_v], add=True)
        return 0

    lax.fori_loop(0, EPW // DCH, chunk_body, 0)
    plsc.subcore_barrier()

    for k in range(RPT // DCH):
        r0 = sid * RPT + k * DCH
        pltpu.sync_copy(deg_sh.at[pl.ds(r0, DCH)], buf_v)
        pltpu.sync_copy(buf_v, degp_hbm.at[cid, pl.ds(r0, DCH)])


_sc_deg = pl.kernel(
    _sc_deg_body,
    out_type=jax.ShapeDtypeStruct((NC, NPAD, D), jnp.float32),
    mesh=_mesh,
    scratch_types=[
        pltpu.VMEM((DCH,), jnp.int32),
        pltpu.VMEM((DCH, D), jnp.float32),
        pltpu.VMEM_SHARED((NPAD, D), jnp.float32),
    ],
)


NBUF = 4
CN_TILE = (EP // CH) // NS


def _sc_scatter_body(g_hbm, src_hbm, dst_hbm, accp_hbm,
                     sidx0, didx0, rows0, sidx1, didx1, rows1,
                     sidx2, didx2, rows2, sidx3, didx3, rows3,
                     acc_sh, sem0, sem1, sem2, sem3):
    cid = lax.axis_index("c")
    sid = lax.axis_index("s")
    sidx = (sidx0, sidx1, sidx2, sidx3)
    didx = (didx0, didx1, didx2, didx3)
    rows = (rows0, rows1, rows2, rows3)
    sem = (sem0, sem1, sem2, sem3)

    def load_and_fire(ci, b):
        off = ci * CH
        pltpu.sync_copy(src_hbm.at[pl.ds(off, CH)], sidx[b])
        pltpu.sync_copy(dst_hbm.at[pl.ds(off, CH)], didx[b])
        pltpu.async_copy(g_hbm.at[sidx[b]], rows[b], sem[b])

    def run_edges(base_chunk, nchunks):
        for c0 in range(NBUF - 1):
            load_and_fire(base_chunk + c0, c0)

        def ring_body(cq, _):
            for b in range(NBUF):
                ci = cq * NBUF + b
                nxt = ci + NBUF - 1

                @pl.when(nxt < nchunks)
                def _():
                    load_and_fire(base_chunk + nxt, (b + NBUF - 1) % NBUF)

                pltpu.make_async_copy(g_hbm.at[sidx[b]], rows[b], sem[b]).wait()
                pltpu.sync_copy(rows[b], acc_sh.at[didx[b]], add=True)
            return 0

        lax.fori_loop(0, nchunks // NBUF, ring_body, 0)

    @pl.when(cid == 0)
    def _():
        def zrow(i, _):
            for j in range(D // L):
                rows0[i, pl.ds(j * L, L)] = jnp.zeros((L,), jnp.float32)
            return 0

        lax.fori_loop(0, CH, zrow, 0)
        for k in range(RPT // CH):
            pltpu.sync_copy(rows0, acc_sh.at[pl.ds(sid * RPT + k * CH, CH)])

    plsc.subcore_barrier()

    @pl.when(cid == 0)
    def _():
        run_edges(sid * CN_TILE, CN_TILE)

    plsc.subcore_barrier()

    @pl.when(cid == 0)
    def _():
        for k in range(RPT // CH):
            r0 = sid * RPT + k * CH
            pltpu.sync_copy(acc_sh.at[pl.ds(r0, CH)], rows0)
            pltpu.sync_copy(rows0, accp_hbm.at[pl.ds(r0, CH)])


_sc_scatter = pl.kernel(
    _sc_scatter_body,
    out_type=jax.ShapeDtypeStruct((NPAD, D), jnp.float32),
    mesh=_mesh,
    scratch_types=(
        [pltpu.VMEM((CH,), jnp.int32),
         pltpu.VMEM((CH,), jnp.int32),
         pltpu.VMEM((CH, D), jnp.float32)] * NBUF
        + [pltpu.VMEM_SHARED((NPAD, D), jnp.float32)]
        + [pltpu.SemaphoreType.DMA] * NBUF
    ),
)


def _tc_first_body(x_ref, w_ref, degp_ref, g_ref, dinv_ref):
    deg_col = degp_ref[0, :, 0:1] + degp_ref[1, :, 0:1] + 1.0
    dinv = lax.rsqrt(deg_col)
    dinv_ref[...] = dinv
    xw = jnp.dot(x_ref[...], w_ref[...], preferred_element_type=jnp.float32)
    g_ref[0:N, :] = dinv[0:N] * xw
    g_ref[N:NPAD, :] = jnp.zeros((NPAD - N, D), jnp.float32)


_tc_first = pl.pallas_call(
    _tc_first_body,
    out_shape=(
        jax.ShapeDtypeStruct((NPAD, D), jnp.float32),
        jax.ShapeDtypeStruct((NPAD, 1), jnp.float32),
    ),
)


def _pool(h10k, batch_ref):
    iota = lax.broadcasted_iota(jnp.int32, (G, N), 0)
    onehot = jnp.where(batch_ref[...] == iota, 1.0, 0.0).astype(jnp.float32)
    s = jnp.dot(onehot, h10k, preferred_element_type=jnp.float32)
    cnt = jnp.dot(onehot, jnp.ones((N, 1), jnp.float32),
                  preferred_element_type=jnp.float32)
    return s / jnp.maximum(cnt, 1.0)


def _conv_out(accp_ref, g_ref, dinv_ref, b_ref):
    acc = accp_ref[...] + g_ref[...]
    pre = dinv_ref[...] * acc + b_ref[...]
    return jnp.where(pre >= 0.0, pre, 0.01 * pre)


def _tc_mid_body(accp_ref, g_ref, dinv_ref, wn_ref, b_ref, batch_ref,
                 gn_ref, p_ref):
    h = _conv_out(accp_ref, g_ref, dinv_ref, b_ref)
    p_ref[...] = _pool(h[0:N, :], batch_ref)
    hw = jnp.dot(h, wn_ref[...], preferred_element_type=jnp.float32)
    gn_ref[0:N, :] = dinv_ref[0:N] * hw[0:N, :]
    gn_ref[N:NPAD, :] = jnp.zeros((NPAD - N, D), jnp.float32)


_tc_mid = pl.pallas_call(
    _tc_mid_body,
    out_shape=(
        jax.ShapeDtypeStruct((NPAD, D), jnp.float32),
        jax.ShapeDtypeStruct((G, D), jnp.float32),
    ),
)


def _tc_last_body(accp_ref, g_ref, dinv_ref, b_ref, batch_ref, p1_ref,
                  p2_ref, out_ref):
    h = _conv_out(accp_ref, g_ref, dinv_ref, b_ref)
    p3 = _pool(h[0:N, :], batch_ref)
    out_ref[...] = (p1_ref[...] + p2_ref[...] + p3) * (1.0 / 3.0)


_tc_last = pl.pallas_call(
    _tc_last_body,
    out_shape=jax.ShapeDtypeStruct((G, D), jnp.float32),
)


def kernel(x, edge_index, batch, W1, b1, W2, b2, W3, b3):
    pad = jnp.full((EP - E,), N, dtype=jnp.int32)
    srcp = jnp.concatenate([edge_index[0], pad])
    dstp = jnp.concatenate([edge_index[1], pad])
    batch2d = batch.reshape(1, N)
    b1r = b1.reshape(1, D)
    b2r = b2.reshape(1, D)
    b3r = b3.reshape(1, D)

    degp = _sc_deg(dstp)
    g1, dinv = _tc_first(x, W1, degp)
    acc1 = _sc_scatter(g1, srcp, dstp)
    g2, p1 = _tc_mid(acc1, g1, dinv, W2, b1r, batch2d)
    acc2 = _sc_scatter(g2, srcp, dstp)
    g3, p2 = _tc_mid(acc2, g2, dinv, W3, b2r, batch2d)
    acc3 = _sc_scatter(g3, srcp, dstp)
    merge = _tc_last(acc3, g3, dinv, b3r, batch2d, p1, p2)
    return (merge, 0)


# speedup vs baseline: 1.4279x; 1.4279x over previous
import jax
import jax.numpy as jnp
from jax import lax
from jax.experimental import pallas as pl
from jax.experimental.pallas import tpu as pltpu
from jax.experimental.pallas import tpu_sc as plsc

N = 10000
E = 320000
G = 64
D = 128

NC = 2
NS = 16
NW = NC * NS
CH = 80
EPW = 10240
EP = NW * EPW
NPAD = 10240
RPT = NPAD // NS
L = 16

_mesh = plsc.VectorSubcoreMesh(
    core_axis_name="c", subcore_axis_name="s", num_cores=NC, num_subcores=NS
)


DCH = 128


def _sc_deg_body(dst_hbm, degp_hbm, idx_v, buf_v, deg_sh):
    cid = lax.axis_index("c")
    sid = lax.axis_index("s")
    wid = sid * NC + cid
    base = wid * EPW

    def fill(val):
        def body(i, _):
            for j in range(D // L):
                buf_v[i, pl.ds(j * L, L)] = jnp.full((L,), val, jnp.float32)
            return 0
        return body

    lax.fori_loop(0, DCH, fill(0.0), 0)
    for k in range(RPT // DCH):
        pltpu.sync_copy(buf_v, deg_sh.at[pl.ds(sid * RPT + k * DCH, DCH)])
    plsc.subcore_barrier()
    lax.fori_loop(0, DCH, fill(1.0), 0)

    def chunk_body(ci, _):
        pltpu.sync_copy(dst_hbm.at[pl.ds(base + ci * DCH, DCH)], idx_v)
        pltpu.sync_copy(buf_v, deg_sh.at[idx_v], add=True)
        return 0

    lax.fori_loop(0, EPW // DCH, chunk_body, 0)
    plsc.subcore_barrier()

    for k in range(RPT // DCH):
        r0 = sid * RPT + k * DCH
        pltpu.sync_copy(deg_sh.at[pl.ds(r0, DCH)], buf_v)
        pltpu.sync_copy(buf_v, degp_hbm.at[cid, pl.ds(r0, DCH)])


_sc_deg = pl.kernel(
    _sc_deg_body,
    out_type=jax.ShapeDtypeStruct((NC, NPAD, D), jnp.float32),
    mesh=_mesh,
    scratch_types=[
        pltpu.VMEM((DCH,), jnp.int32),
        pltpu.VMEM((DCH, D), jnp.float32),
        pltpu.VMEM_SHARED((NPAD, D), jnp.float32),
    ],
)


NBUF = 4
CN_FAST = 160
CN_SLOW = 96


def _sc_scatter_body(g_hbm, src_hbm, dst_hbm, accp_hbm,
                     sidx0, didx0, rows0, sidx1, didx1, rows1,
                     sidx2, didx2, rows2, sidx3, didx3, rows3,
                     acc_sh, sem0, sem1, sem2, sem3):
    cid = lax.axis_index("c")
    sid = lax.axis_index("s")
    sidx = (sidx0, sidx1, sidx2, sidx3)
    didx = (didx0, didx1, didx2, didx3)
    rows = (rows0, rows1, rows2, rows3)
    sem = (sem0, sem1, sem2, sem3)

    def load_and_fire(ci, b):
        off = ci * CH
        pltpu.sync_copy(src_hbm.at[pl.ds(off, CH)], sidx[b])
        pltpu.sync_copy(dst_hbm.at[pl.ds(off, CH)], didx[b])
        pltpu.async_copy(g_hbm.at[sidx[b]], rows[b], sem[b])

    def run_edges(base_chunk, nchunks):
        for c0 in range(NBUF - 1):
            load_and_fire(base_chunk + c0, c0)

        def ring_body(cq, _):
            for b in range(NBUF):
                ci = cq * NBUF + b
                nxt = ci + NBUF - 1

                @pl.when(nxt < nchunks)
                def _():
                    load_and_fire(base_chunk + nxt, (b + NBUF - 1) % NBUF)

                pltpu.make_async_copy(g_hbm.at[sidx[b]], rows[b], sem[b]).wait()
                pltpu.sync_copy(rows[b], acc_sh.at[didx[b]], add=True)
            return 0

        lax.fori_loop(0, nchunks // NBUF, ring_body, 0)

    def zrow(i, _):
        for j in range(D // L):
            rows0[i, pl.ds(j * L, L)] = jnp.zeros((L,), jnp.float32)
        return 0

    lax.fori_loop(0, CH, zrow, 0)
    for k in range(RPT // CH):
        pltpu.sync_copy(rows0, acc_sh.at[pl.ds(sid * RPT + k * CH, CH)])

    plsc.subcore_barrier()

    @pl.when(cid == 0)
    def _():
        run_edges(sid * CN_FAST, CN_FAST)

    @pl.when(cid == 1)
    def _():
        run_edges(NS * CN_FAST + sid * CN_SLOW, CN_SLOW)

    plsc.subcore_barrier()

    for k in range(RPT // CH):
        r0 = sid * RPT + k * CH
        pltpu.sync_copy(acc_sh.at[pl.ds(r0, CH)], rows0)
        pltpu.sync_copy(rows0, accp_hbm.at[cid, pl.ds(r0, CH)])


_sc_scatter = pl.kernel(
    _sc_scatter_body,
    out_type=jax.ShapeDtypeStruct((NC, NPAD, D), jnp.float32),
    mesh=_mesh,
    scratch_types=(
        [pltpu.VMEM((CH,), jnp.int32),
         pltpu.VMEM((CH,), jnp.int32),
         pltpu.VMEM((CH, D), jnp.float32)] * NBUF
        + [pltpu.VMEM_SHARED((NPAD, D), jnp.float32)]
        + [pltpu.SemaphoreType.DMA] * NBUF
    ),
)


def _tc_first_body(x_ref, w_ref, degp_ref, g_ref, dinv_ref):
    deg_col = degp_ref[0, :, 0:1] + degp_ref[1, :, 0:1] + 1.0
    dinv = lax.rsqrt(deg_col)
    dinv_ref[...] = dinv
    xw = jnp.dot(x_ref[...], w_ref[...], preferred_element_type=jnp.float32)
    g_ref[0:N, :] = dinv[0:N] * xw
    g_ref[N:NPAD, :] = jnp.zeros((NPAD - N, D), jnp.float32)


_tc_first = pl.pallas_call(
    _tc_first_body,
    out_shape=(
        jax.ShapeDtypeStruct((NPAD, D), jnp.float32),
        jax.ShapeDtypeStruct((NPAD, 1), jnp.float32),
    ),
)


def _pool(h10k, batch_ref):
    iota = lax.broadcasted_iota(jnp.int32, (G, N), 0)
    onehot = jnp.where(batch_ref[...] == iota, 1.0, 0.0).astype(jnp.float32)
    s = jnp.dot(onehot, h10k, preferred_element_type=jnp.float32)
    cnt = jnp.dot(onehot, jnp.ones((N, 1), jnp.float32),
                  preferred_element_type=jnp.float32)
    return s / jnp.maximum(cnt, 1.0)


def _conv_out(accp_ref, g_ref, dinv_ref, b_ref):
    acc = accp_ref[0] + accp_ref[1] + g_ref[...]
    pre = dinv_ref[...] * acc + b_ref[...]
    return jnp.where(pre >= 0.0, pre, 0.01 * pre)


def _tc_mid_body(accp_ref, g_ref, dinv_ref, wn_ref, b_ref, batch_ref,
                 gn_ref, p_ref):
    h = _conv_out(accp_ref, g_ref, dinv_ref, b_ref)
    p_ref[...] = _pool(h[0:N, :], batch_ref)
    hw = jnp.dot(h, wn_ref[...], preferred_element_type=jnp.float32)
    gn_ref[0:N, :] = dinv_ref[0:N] * hw[0:N, :]
    gn_ref[N:NPAD, :] = jnp.zeros((NPAD - N, D), jnp.float32)


_tc_mid = pl.pallas_call(
    _tc_mid_body,
    out_shape=(
        jax.ShapeDtypeStruct((NPAD, D), jnp.float32),
        jax.ShapeDtypeStruct((G, D), jnp.float32),
    ),
)


def _tc_last_body(accp_ref, g_ref, dinv_ref, b_ref, batch_ref, p1_ref,
                  p2_ref, out_ref):
    h = _conv_out(accp_ref, g_ref, dinv_ref, b_ref)
    p3 = _pool(h[0:N, :], batch_ref)
    out_ref[...] = (p1_ref[...] + p2_ref[...] + p3) * (1.0 / 3.0)


_tc_last = pl.pallas_call(
    _tc_last_body,
    out_shape=jax.ShapeDtypeStruct((G, D), jnp.float32),
)


def kernel(x, edge_index, batch, W1, b1, W2, b2, W3, b3):
    pad = jnp.full((EP - E,), N, dtype=jnp.int32)
    srcp = jnp.concatenate([edge_index[0], pad])
    dstp = jnp.concatenate([edge_index[1], pad])
    batch2d = batch.reshape(1, N)
    b1r = b1.reshape(1, D)
    b2r = b2.reshape(1, D)
    b3r = b3.reshape(1, D)

    degp = _sc_deg(dstp)
    g1, dinv = _tc_first(x, W1, degp)
    acc1 = _sc_scatter(g1, srcp, dstp)
    g2, p1 = _tc_mid(acc1, g1, dinv, W2, b1r, batch2d)
    acc2 = _sc_scatter(g2, srcp, dstp)
    g3, p2 = _tc_mid(acc2, g2, dinv, W3, b2r, batch2d)
    acc3 = _sc_scatter(g3, srcp, dstp)
    merge = _tc_last(acc3, g3, dinv, b3r, batch2d, p1, p2)
    return (merge, 0)
